# baseline (speedup 1.0000x reference)
import functools

import jax
import jax.numpy as jnp
from jax import lax
from jax.experimental import pallas as pl
from jax.experimental.pallas import tpu as pltpu
from jax.experimental.pallas import tpu_sc as plsc

_D = 128
_TSTRIDE = 129
_L = 64
_VROWS = 513
_NC, _NS, _LANES = 2, 16, 16
_NW = _NC * _NS
_GROUPS = 800
_GP = _GROUPS // _NW
_SP = _GROUPS * _LANES
_GSLICE = (_VROWS - 1) * _TSTRIDE + 1
_OSLICE = (_LANES - 1) * _D + 1


def _ln_body(tab_ref, w_ref, b_ref, out_ref):
    x = tab_ref[...]
    mu = jnp.mean(x, axis=-1, keepdims=True)
    var = jnp.mean((x - mu) ** 2, axis=-1, keepdims=True)
    out_ref[...] = (x - mu) / jnp.sqrt(var + 1e-5) * w_ref[...] + b_ref[...]


def _ln_table(table, ln_w, ln_b):
    return pl.pallas_call(
        _ln_body,
        out_shape=jax.ShapeDtypeStruct((_VROWS, _D), jnp.float32),
    )(table, ln_w.reshape(1, _D), ln_b.reshape(1, _D))


def _mm_body(x_ref, w_ref, o_ref):
    o_ref[...] = lax.dot_general(
        x_ref[...], w_ref[...], (((1,), (1,)), ((), ())),
        preferred_element_type=jnp.float32)


def _mm(x_cm, w):
    gblk = 32
    return pl.pallas_call(
        _mm_body,
        grid=(_GROUPS // gblk,),
        in_specs=[
            pl.BlockSpec((gblk, _D, _LANES), lambda i: (i, 0, 0)),
            pl.BlockSpec((_D, _D), lambda i: (0, 0)),
        ],
        out_specs=pl.BlockSpec((gblk, _LANES, _D), lambda i: (i, 0, 0)),
        out_shape=jax.ShapeDtypeStruct((_GROUPS, _LANES, _D), jnp.float32),
    )(x_cm, w)


def _sc_gather_max(tab_flat, idx_flat):
    mesh = plsc.VectorSubcoreMesh(core_axis_name="c", subcore_axis_name="s")

    @functools.partial(
        pl.kernel,
        out_type=jax.ShapeDtypeStruct((_SP * _D,), jnp.float32),
        mesh=mesh,
        scratch_types=[
            pltpu.VMEM((_VROWS * _TSTRIDE,), jnp.float32),
            pltpu.VMEM((_LANES * _L,), jnp.int32),
            pltpu.VMEM((_LANES * _L,), jnp.int32),
            pltpu.VMEM((_LANES * _D,), jnp.float32),
            pltpu.VMEM((_LANES * _D,), jnp.float32),
            pltpu.SemaphoreType.DMA,
            pltpu.SemaphoreType.DMA,
        ],
        compiler_params=pltpu.CompilerParams(needs_layout_passes=False),
    )
    def k(tab_hbm, idx_hbm, out_hbm, tab_v, ibuf0, ibuf1, obuf0, obuf1,
          sem_i, sem_o):
        ibufs = (ibuf0, ibuf1)
        obufs = (obuf0, obuf1)
        wid = lax.axis_index("s") * _NC + lax.axis_index("c")
        g0 = wid * _GP

        def issue_idx(g, b):
            pltpu.make_async_copy(
                idx_hbm.at[pl.ds((g0 + g) * (_LANES * _L), _LANES * _L)],
                ibufs[b], sem_i).start()

        def wait_idx(b):
            pltpu.make_async_copy(
                idx_hbm.at[pl.ds(0, _LANES * _L)], ibufs[b], sem_i).wait()

        def issue_out(g, b):
            pltpu.make_async_copy(
                obufs[b],
                out_hbm.at[pl.ds((g0 + g) * (_LANES * _D), _LANES * _D)],
                sem_o).start()

        def wait_out():
            pltpu.make_async_copy(
                obufs[0],
                out_hbm.at[pl.ds(0, _LANES * _D)], sem_o).wait()

        def compute(b):
            ib = ibufs[b]
            ob = obufs[b]
            base0 = ib[pl.ds(0, _LANES)] * _TSTRIDE
            base1 = ib[pl.ds(_LANES, _LANES)] * _TSTRIDE
            for dc in range(4):
                d0 = dc * 32
                accs = tuple(plsc.load_gather(tab_v, [base0 + (d0 + j)])
                             for j in range(32))

                def l_body(l, carry):
                    accs, base = carry
                    accs = tuple(
                        jnp.maximum(a, plsc.load_gather(tab_v, [base + (d0 + j)]))
                        for j, a in enumerate(accs))
                    nl = jnp.minimum(l + 1, _L - 1)
                    nbase = ib[pl.ds(nl * _LANES, _LANES)] * _TSTRIDE
                    return accs, nbase

                accs, _ = lax.fori_loop(1, _L, l_body, (accs, base1))
                for j in range(32):
                    ob[pl.ds((d0 + j) * _LANES, _LANES)] = accs[j]

        issue_idx(0, 0)
        pltpu.sync_copy(tab_hbm, tab_v)
        wait_idx(0)
        issue_idx(1, 1)
        compute(0)
        issue_out(0, 0)
        wait_idx(1)
        issue_idx(2, 0)
        compute(1)
        issue_out(1, 1)

        def pair_body(p, carry):
            g = p * 2
            for b in range(2):
                gg = g + b
                wait_idx(b)
                issue_idx(gg + 1, 1 - b)
                wait_out()
                compute(b)
                issue_out(gg, b)
            return carry

        lax.fori_loop(1, _GP // 2, pair_body, 0)
        wait_idx(0)
        wait_out()
        compute(0)
        issue_out(_GP - 1, 0)
        wait_out()
        wait_out()

    return k(tab_flat, idx_flat)


def kernel(input, table, ln_w, ln_b, W):
    B, N, L = input.shape
    S = B * N
    idx = input.reshape(S, L).astype(jnp.int32)
    idx = jnp.pad(idx, ((0, _SP - S), (0, 0)))
    idx_t = idx.reshape(_GROUPS, _LANES, L).transpose(0, 2, 1)
    ln_tab = _ln_table(table, ln_w, ln_b)
    ln_tab = jnp.pad(ln_tab, ((0, 0), (0, _TSTRIDE - _D)))
    x_flat = _sc_gather_max(ln_tab.reshape(-1), idx_t.reshape(-1))
    Y = _mm(x_flat.reshape(_GROUPS, _D, _LANES), W)
    return Y.reshape(_SP, _D)[:S].reshape(B, N, _D)


# speedup vs baseline: 1.0042x; 1.0042x over previous
import functools

import jax
import jax.numpy as jnp
from jax import lax
from jax.experimental import pallas as pl
from jax.experimental.pallas import tpu as pltpu
from jax.experimental.pallas import tpu_sc as plsc

_D = 128
_TSTRIDE = 129
_L = 64
_VROWS = 513
_NC, _NS, _LANES = 2, 16, 16
_NW = _NC * _NS
_GROUPS = 800
_GP = _GROUPS // _NW
_SP = _GROUPS * _LANES
_GSLICE = (_VROWS - 1) * _TSTRIDE + 1
_OSLICE = (_LANES - 1) * _D + 1


def _ln_body(tab_ref, w_ref, b_ref, out_ref):
    x = tab_ref[...]
    mu = jnp.mean(x, axis=-1, keepdims=True)
    var = jnp.mean((x - mu) ** 2, axis=-1, keepdims=True)
    out_ref[...] = (x - mu) / jnp.sqrt(var + 1e-5) * w_ref[...] + b_ref[...]


def _ln_table(table, ln_w, ln_b):
    return pl.pallas_call(
        _ln_body,
        out_shape=jax.ShapeDtypeStruct((_VROWS, _D), jnp.float32),
    )(table, ln_w.reshape(1, _D), ln_b.reshape(1, _D))


def _mm_body(x_ref, w_ref, o_ref):
    o_ref[...] = lax.dot_general(
        x_ref[...], w_ref[...], (((1,), (1,)), ((), ())),
        preferred_element_type=jnp.float32)


def _mm(x, w):
    rows = x.shape[0]
    blk = 512
    return pl.pallas_call(
        _mm_body,
        grid=(rows // blk,),
        in_specs=[
            pl.BlockSpec((blk, _D), lambda i: (i, 0)),
            pl.BlockSpec((_D, _D), lambda i: (0, 0)),
        ],
        out_specs=pl.BlockSpec((blk, _D), lambda i: (i, 0)),
        out_shape=jax.ShapeDtypeStruct((rows, _D), jnp.float32),
    )(x, w)


def _sc_gather_max(tab_flat, idx_flat):
    mesh = plsc.VectorSubcoreMesh(core_axis_name="c", subcore_axis_name="s")

    @functools.partial(
        pl.kernel,
        out_type=jax.ShapeDtypeStruct((_SP * _D,), jnp.float32),
        mesh=mesh,
        scratch_types=[
            pltpu.VMEM((_VROWS * _TSTRIDE,), jnp.float32),
            pltpu.VMEM((_LANES * _L,), jnp.int32),
            pltpu.VMEM((_LANES * _L,), jnp.int32),
            pltpu.VMEM((_LANES * _D,), jnp.float32),
            pltpu.VMEM((_LANES * _D,), jnp.float32),
            pltpu.SemaphoreType.DMA,
            pltpu.SemaphoreType.DMA,
        ],
        compiler_params=pltpu.CompilerParams(needs_layout_passes=False),
    )
    def k(tab_hbm, idx_hbm, out_hbm, tab_v, ibuf0, ibuf1, obuf0, obuf1,
          sem_i, sem_o):
        ibufs = (ibuf0, ibuf1)
        obufs = (obuf0, obuf1)
        wid = lax.axis_index("s") * _NC + lax.axis_index("c")
        g0 = wid * _GP

        def issue_idx(g, b):
            pltpu.make_async_copy(
                idx_hbm.at[pl.ds((g0 + g) * (_LANES * _L), _LANES * _L)],
                ibufs[b], sem_i).start()

        def wait_idx(b):
            pltpu.make_async_copy(
                idx_hbm.at[pl.ds(0, _LANES * _L)], ibufs[b], sem_i).wait()

        def issue_out(g, b):
            pltpu.make_async_copy(
                obufs[b],
                out_hbm.at[pl.ds((g0 + g) * (_LANES * _D), _LANES * _D)],
                sem_o).start()

        def wait_out():
            pltpu.make_async_copy(
                obufs[0],
                out_hbm.at[pl.ds(0, _LANES * _D)], sem_o).wait()

        def compute(b):
            ib = ibufs[b]
            ob = obufs[b]
            base0 = ib[pl.ds(0, _LANES)] * _TSTRIDE
            base1 = ib[pl.ds(_LANES, _LANES)] * _TSTRIDE
            for dc in range(4):
                d0 = dc * 32
                accs = tuple(plsc.load_gather(tab_v, [base0 + (d0 + j)])
                             for j in range(32))

                def l_body(l, carry):
                    accs, base = carry
                    accs = tuple(
                        jnp.maximum(a, plsc.load_gather(tab_v, [base + (d0 + j)]))
                        for j, a in enumerate(accs))
                    nl = jnp.minimum(l + 1, _L - 1)
                    nbase = ib[pl.ds(nl * _LANES, _LANES)] * _TSTRIDE
                    return accs, nbase

                accs, _ = lax.fori_loop(1, _L, l_body, (accs, base1))
                for j in range(32):
                    ob[pl.ds((d0 + j) * _LANES, _LANES)] = accs[j]

        issue_idx(0, 0)
        pltpu.sync_copy(tab_hbm, tab_v)
        wait_idx(0)
        issue_idx(1, 1)
        compute(0)
        issue_out(0, 0)
        wait_idx(1)
        issue_idx(2, 0)
        compute(1)
        issue_out(1, 1)

        def pair_body(p, carry):
            g = p * 2
            for b in range(2):
                gg = g + b
                wait_idx(b)
                issue_idx(gg + 1, 1 - b)
                wait_out()
                compute(b)
                issue_out(gg, b)
            return carry

        lax.fori_loop(1, _GP // 2, pair_body, 0)
        wait_idx(0)
        wait_out()
        compute(0)
        issue_out(_GP - 1, 0)
        wait_out()
        wait_out()

    return k(tab_flat, idx_flat)


def kernel(input, table, ln_w, ln_b, W):
    B, N, L = input.shape
    S = B * N
    idx = input.reshape(S, L).astype(jnp.int32)
    idx = jnp.pad(idx, ((0, _SP - S), (0, 0)))
    idx_t = idx.reshape(_GROUPS, _LANES, L).transpose(0, 2, 1)
    ln_tab = _ln_table(table, ln_w, ln_b)
    ln_tab = jnp.pad(ln_tab, ((0, 0), (0, _TSTRIDE - _D)))
    x_flat = _sc_gather_max(ln_tab.reshape(-1), idx_t.reshape(-1))
    x = x_flat.reshape(_GROUPS, _D, _LANES).transpose(0, 2, 1)
    Y = _mm(x.reshape(_SP, _D), W)
    return Y[:S].reshape(B, N, _D)


# speedup vs baseline: 1.0289x; 1.0246x over previous
import functools

import jax
import jax.numpy as jnp
from jax import lax
from jax.experimental import pallas as pl
from jax.experimental.pallas import tpu as pltpu
from jax.experimental.pallas import tpu_sc as plsc

_D = 128
_TSTRIDE = 129
_L = 64
_VROWS = 513
_NC, _NS, _LANES = 2, 16, 16
_NW = _NC * _NS
_GROUPS = 800
_GP = _GROUPS // _NW
_SP = _GROUPS * _LANES
_OSTRIDE = 129


def _ln_body(tab_ref, w_ref, b_ref, out_ref):
    x = tab_ref[...]
    mu = jnp.mean(x, axis=-1, keepdims=True)
    var = jnp.mean((x - mu) ** 2, axis=-1, keepdims=True)
    out_ref[...] = (x - mu) / jnp.sqrt(var + 1e-5) * w_ref[...] + b_ref[...]


def _ln_table(table, ln_w, ln_b):
    return pl.pallas_call(
        _ln_body,
        out_shape=jax.ShapeDtypeStruct((_VROWS, _D), jnp.float32),
    )(table, ln_w.reshape(1, _D), ln_b.reshape(1, _D))


def _mm_body(x_ref, w_ref, o_ref):
    o_ref[...] = lax.dot_general(
        x_ref[...], w_ref[...], (((1,), (1,)), ((), ())),
        preferred_element_type=jnp.float32)


def _mm(x, w):
    blk = 512
    return pl.pallas_call(
        _mm_body,
        grid=(_SP // blk,),
        in_specs=[
            pl.BlockSpec((blk, _D), lambda i: (i, 0)),
            pl.BlockSpec((_D, _D), lambda i: (0, 0)),
        ],
        out_specs=pl.BlockSpec((blk, _D), lambda i: (i, 0)),
        out_shape=jax.ShapeDtypeStruct((_SP, _D), jnp.float32),
    )(x, w)


def _sc_gather_max(tab_flat, idx_flat):
    mesh = plsc.VectorSubcoreMesh(core_axis_name="c", subcore_axis_name="s")

    @functools.partial(
        pl.kernel,
        out_type=jax.ShapeDtypeStruct((_SP * _OSTRIDE,), jnp.float32),
        mesh=mesh,
        scratch_types=[
            pltpu.VMEM((_VROWS * _TSTRIDE,), jnp.float32),
            pltpu.VMEM((_LANES * _L,), jnp.int32),
            pltpu.VMEM((_LANES * _L,), jnp.int32),
            pltpu.VMEM((_LANES * _OSTRIDE,), jnp.float32),
            pltpu.VMEM((_LANES * _OSTRIDE,), jnp.float32),
            pltpu.SemaphoreType.DMA,
            pltpu.SemaphoreType.DMA,
        ],
        compiler_params=pltpu.CompilerParams(needs_layout_passes=False),
    )
    def k(tab_hbm, idx_hbm, out_hbm, tab_v, ibuf0, ibuf1, obuf0, obuf1,
          sem_i, sem_o):
        ibufs = (ibuf0, ibuf1)
        obufs = (obuf0, obuf1)
        wid = lax.axis_index("s") * _NC + lax.axis_index("c")
        g0 = wid * _GP
        row_base = lax.iota(jnp.int32, 16) * _OSTRIDE

        def issue_idx(g, b):
            pltpu.make_async_copy(
                idx_hbm.at[pl.ds((g0 + g) * (_LANES * _L), _LANES * _L)],
                ibufs[b], sem_i).start()

        def wait_idx(b):
            pltpu.make_async_copy(
                idx_hbm.at[pl.ds(0, _LANES * _L)], ibufs[b], sem_i).wait()

        def issue_out(g, b):
            pltpu.make_async_copy(
                obufs[b],
                out_hbm.at[pl.ds((g0 + g) * (_LANES * _OSTRIDE),
                                 _LANES * _OSTRIDE)],
                sem_o).start()

        def wait_out():
            pltpu.make_async_copy(
                obufs[0],
                out_hbm.at[pl.ds(0, _LANES * _OSTRIDE)], sem_o).wait()

        def compute(b):
            ib = ibufs[b]
            ob = obufs[b]
            base0 = ib[pl.ds(0, _LANES)] * _TSTRIDE
            base1 = ib[pl.ds(_LANES, _LANES)] * _TSTRIDE
            for dc in range(4):
                d0 = dc * 32
                accs = tuple(plsc.load_gather(tab_v, [base0 + (d0 + j)])
                             for j in range(32))

                def l_body(l, carry):
                    accs, base = carry
                    accs = tuple(
                        jnp.maximum(a, plsc.load_gather(tab_v, [base + (d0 + j)]))
                        for j, a in enumerate(accs))
                    nl = jnp.minimum(l + 1, _L - 1)
                    nbase = ib[pl.ds(nl * _LANES, _LANES)] * _TSTRIDE
                    return accs, nbase

                accs, _ = lax.fori_loop(1, _L, l_body, (accs, base1))
                for j in range(32):
                    plsc.store_scatter(ob, [row_base + (d0 + j)], accs[j])

        issue_idx(0, 0)
        pltpu.sync_copy(tab_hbm, tab_v)
        wait_idx(0)
        issue_idx(1, 1)
        compute(0)
        issue_out(0, 0)
        wait_idx(1)
        issue_idx(2, 0)
        compute(1)
        issue_out(1, 1)

        def pair_body(p, carry):
            g = p * 2
            for b in range(2):
                gg = g + b
                wait_idx(b)
                issue_idx(gg + 1, 1 - b)
                wait_out()
                compute(b)
                issue_out(gg, b)
            return carry

        lax.fori_loop(1, _GP // 2, pair_body, 0)
        wait_idx(0)
        wait_out()
        compute(0)
        issue_out(_GP - 1, 0)
        wait_out()
        wait_out()

    return k(tab_flat, idx_flat)


def kernel(input, table, ln_w, ln_b, W):
    B, N, L = input.shape
    S = B * N
    idx = input.reshape(S, L).astype(jnp.int32)
    idx = jnp.pad(idx, ((0, _SP - S), (0, 0)))
    idx_t = idx.reshape(_GROUPS, _LANES, L).transpose(0, 2, 1)
    ln_tab = _ln_table(table, ln_w, ln_b)
    ln_tab = jnp.pad(ln_tab, ((0, 0), (0, _TSTRIDE - _D)))
    x_flat = _sc_gather_max(ln_tab.reshape(-1), idx_t.reshape(-1))
    Y = _mm(x_flat.reshape(_SP, _OSTRIDE), W)
    return Y[:S].reshape(B, N, _D)


# speedup vs baseline: 1.0492x; 1.0197x over previous
import functools

import jax
import jax.numpy as jnp
from jax import lax
from jax.experimental import pallas as pl
from jax.experimental.pallas import tpu as pltpu
from jax.experimental.pallas import tpu_sc as plsc

_D = 128
_TSTRIDE = 129
_L = 64
_VROWS = 513
_NC, _NS, _LANES = 2, 16, 16
_NW = _NC * _NS
_GROUPS = 800
_GP = _GROUPS // _NW
_SP = _GROUPS * _LANES
_OSTRIDE = 129


def _ln_body(tab_ref, w_ref, b_ref, out_ref):
    x = tab_ref[...]
    mu = jnp.mean(x, axis=-1, keepdims=True)
    var = jnp.mean((x - mu) ** 2, axis=-1, keepdims=True)
    out_ref[...] = (x - mu) / jnp.sqrt(var + 1e-5) * w_ref[...] + b_ref[...]


def _ln_table(table, ln_w, ln_b):
    return pl.pallas_call(
        _ln_body,
        out_shape=jax.ShapeDtypeStruct((_VROWS, _D), jnp.float32),
    )(table, ln_w.reshape(1, _D), ln_b.reshape(1, _D))


def _mm_body(x_ref, w_ref, o_ref):
    o_ref[...] = lax.dot_general(
        x_ref[...], w_ref[...], (((1,), (1,)), ((), ())),
        preferred_element_type=jnp.float32)


def _mm(x, w):
    blk = 512
    return pl.pallas_call(
        _mm_body,
        grid=(_SP // blk,),
        in_specs=[
            pl.BlockSpec((blk, _D), lambda i: (i, 0)),
            pl.BlockSpec((_D, _D), lambda i: (0, 0)),
        ],
        out_specs=pl.BlockSpec((blk, _D), lambda i: (i, 0)),
        out_shape=jax.ShapeDtypeStruct((_SP, _D), jnp.float32),
    )(x, w)


def _sc_gather_max(tab_flat, idx_flat):
    mesh = plsc.VectorSubcoreMesh(core_axis_name="c", subcore_axis_name="s")

    @functools.partial(
        pl.kernel,
        out_type=jax.ShapeDtypeStruct((_SP, _D), jnp.float32),
        mesh=mesh,
        scratch_types=[
            pltpu.VMEM((_VROWS * _TSTRIDE,), jnp.float32),
            pltpu.VMEM((_LANES * _L,), jnp.int32),
            pltpu.VMEM((_LANES * _L,), jnp.int32),
            pltpu.VMEM((_LANES, _OSTRIDE), jnp.float32),
            pltpu.VMEM((_LANES, _OSTRIDE), jnp.float32),
            pltpu.SemaphoreType.DMA,
            pltpu.SemaphoreType.DMA,
        ],
        compiler_params=pltpu.CompilerParams(needs_layout_passes=False),
    )
    def k(tab_hbm, idx_hbm, out_hbm, tab_v, ibuf0, ibuf1, obuf0, obuf1,
          sem_i, sem_o):
        ibufs = (ibuf0, ibuf1)
        obufs = (obuf0, obuf1)
        wid = lax.axis_index("s") * _NC + lax.axis_index("c")
        g0 = wid * _GP
        lane_ids = lax.iota(jnp.int32, 16)

        def issue_idx(g, b):
            pltpu.make_async_copy(
                idx_hbm.at[pl.ds((g0 + g) * (_LANES * _L), _LANES * _L)],
                ibufs[b], sem_i).start()

        def wait_idx(b):
            pltpu.make_async_copy(
                idx_hbm.at[pl.ds(0, _LANES * _L)], ibufs[b], sem_i).wait()

        def issue_out(g, b):
            pltpu.make_async_copy(
                obufs[b].at[:, pl.ds(0, _D)],
                out_hbm.at[pl.ds((g0 + g) * _LANES, _LANES), :],
                sem_o).start()

        def wait_out():
            pltpu.make_async_copy(
                obufs[0].at[:, pl.ds(0, _D)],
                out_hbm.at[pl.ds(0, _LANES), :], sem_o).wait()

        def compute(b):
            ib = ibufs[b]
            ob = obufs[b]
            base0 = ib[pl.ds(0, _LANES)] * _TSTRIDE
            base1 = ib[pl.ds(_LANES, _LANES)] * _TSTRIDE
            for dc in range(4):
                d0 = dc * 32
                accs = tuple(plsc.load_gather(tab_v, [base0 + (d0 + j)])
                             for j in range(32))

                def l_body(l, carry):
                    accs, base = carry
                    accs = tuple(
                        jnp.maximum(a, plsc.load_gather(tab_v, [base + (d0 + j)]))
                        for j, a in enumerate(accs))
                    nl = jnp.minimum(l + 1, _L - 1)
                    nbase = ib[pl.ds(nl * _LANES, _LANES)] * _TSTRIDE
                    return accs, nbase

                accs, _ = lax.fori_loop(1, _L, l_body, (accs, base1))
                for j in range(32):
                    plsc.store_scatter(
                        ob, [lane_ids, jnp.full((16,), d0 + j, jnp.int32)],
                        accs[j])

        issue_idx(0, 0)
        pltpu.sync_copy(tab_hbm, tab_v)
        wait_idx(0)
        issue_idx(1, 1)
        compute(0)
        issue_out(0, 0)
        wait_idx(1)
        issue_idx(2, 0)
        compute(1)
        issue_out(1, 1)

        def pair_body(p, carry):
            g = p * 2
            for b in range(2):
                gg = g + b
                wait_idx(b)
                issue_idx(gg + 1, 1 - b)
                wait_out()
                compute(b)
                issue_out(gg, b)
            return carry

        lax.fori_loop(1, _GP // 2, pair_body, 0)
        wait_idx(0)
        wait_out()
        compute(0)
        issue_out(_GP - 1, 0)
        wait_out()
        wait_out()

    return k(tab_flat, idx_flat)


def kernel(input, table, ln_w, ln_b, W):
    B, N, L = input.shape
    S = B * N
    idx = input.reshape(S, L).astype(jnp.int32)
    idx = jnp.pad(idx, ((0, _SP - S), (0, 0)))
    idx_t = idx.reshape(_GROUPS, _LANES, L).transpose(0, 2, 1)
    ln_tab = _ln_table(table, ln_w, ln_b)
    ln_tab = jnp.pad(ln_tab, ((0, 0), (0, _TSTRIDE - _D)))
    x = _sc_gather_max(ln_tab.reshape(-1), idx_t.reshape(-1))
    Y = _mm(x, W)
    return Y[:S].reshape(B, N, _D)
